# baseline (speedup 1.0000x reference)
import functools

import jax
import jax.numpy as jnp
from jax import lax
from jax.experimental import pallas as pl
from jax.experimental.pallas import tpu as pltpu
from jax.experimental.pallas import tpu_sc as plsc

_N = 10000
_E = 320000
_ND = 10240
_EP = 327680
_NC = 2
_NT = 16
_BLK = 128
_RPT = _ND // _NT
_BLOCKS_ALL = _EP // (_NT * _BLK)
_BLOCKS_HALF = _EP // (_NC * _NT * _BLK)

_CHUNK = 16
_TC_ROWS = 640

_sc_mesh = plsc.VectorSubcoreMesh(core_axis_name="c", subcore_axis_name="s")


def _hist_body(dst_hbm, zeros_hbm, ones_hbm, out_hbm, acc, didx, ones_v,
               sem0, sem1):
    cid = lax.axis_index("c")
    sid = lax.axis_index("s")
    r0 = sid * _RPT
    pltpu.sync_copy(zeros_hbm.at[pl.ds(r0, _RPT)], acc.at[pl.ds(r0, _RPT)])
    pltpu.sync_copy(ones_hbm, ones_v)
    tid = cid * _NT + sid
    blk0 = tid * _BLOCKS_HALF
    pltpu.sync_copy(dst_hbm.at[pl.ds(blk0, _BLOCKS_HALF)], didx)
    plsc.subcore_barrier()

    def body(i, carry):
        @pl.when(i > 0)
        def _():
            pltpu.make_async_copy(ones_v, acc.at[didx.at[0]], sem0).wait()
            pltpu.make_async_copy(ones_v, acc.at[didx.at[0]], sem1).wait()

        pltpu.async_copy(ones_v, acc.at[didx.at[2 * i]], sem0, add=True)
        pltpu.async_copy(ones_v, acc.at[didx.at[2 * i + 1]], sem1, add=True)
        return carry

    lax.fori_loop(0, _BLOCKS_HALF // 2, body, 0)
    pltpu.make_async_copy(ones_v, acc.at[didx.at[0]], sem0).wait()
    pltpu.make_async_copy(ones_v, acc.at[didx.at[0]], sem1).wait()
    plsc.subcore_barrier()
    pltpu.sync_copy(acc.at[pl.ds(r0, _RPT)], out_hbm.at[cid, pl.ds(r0, _RPT)])


_hist_call = pl.kernel(
    _hist_body,
    out_type=jax.ShapeDtypeStruct((_NC, _ND, 128), jnp.float32),
    mesh=_sc_mesh,
    scratch_types=[
        pltpu.VMEM_SHARED((_ND, 128), jnp.float32),
        pltpu.VMEM((_BLOCKS_HALF, _BLK), jnp.int32),
        pltpu.VMEM((_BLK, 128), jnp.float32),
        pltpu.SemaphoreType.DMA,
        pltpu.SemaphoreType.DMA,
    ],
)


def _spmm_body(split_edges, tables_hbm, src_hbm, dst_hbm, out_hbm, acc,
               sidx, didx, rows0, rows1, gsem0, gsem1):
    cid = lax.axis_index("c")
    sid = lax.axis_index("s")
    r0 = sid * _RPT
    if split_edges:
        table = tables_hbm
        nblocks = _BLOCKS_HALF
        blk0 = (cid * _NT + sid) * _BLOCKS_HALF
    else:
        table = tables_hbm.at[cid]
        nblocks = _BLOCKS_ALL
        blk0 = sid * _BLOCKS_ALL
    pltpu.sync_copy(table.at[pl.ds(r0, _RPT)], acc.at[pl.ds(r0, _RPT)])
    plsc.subcore_barrier()

    def chunk_body(ci, carry):
        base = blk0 + ci * _CHUNK
        pltpu.sync_copy(src_hbm.at[pl.ds(base, _CHUNK)], sidx)
        pltpu.sync_copy(dst_hbm.at[pl.ds(base, _CHUNK)], didx)
        pltpu.async_copy(table.at[sidx.at[0]], rows0, gsem0)

        def body(i, c):
            b = 2 * i
            pltpu.async_copy(table.at[sidx.at[b + 1]], rows1, gsem1)
            pltpu.make_async_copy(table.at[sidx.at[b]], rows0, gsem0).wait()
            pltpu.sync_copy(rows0, acc.at[didx.at[b]], add=True)

            @pl.when(b + 2 < _CHUNK)
            def _():
                pltpu.async_copy(table.at[sidx.at[b + 2]], rows0, gsem0)

            pltpu.make_async_copy(table.at[sidx.at[b + 1]], rows1,
                                  gsem1).wait()
            pltpu.sync_copy(rows1, acc.at[didx.at[b + 1]], add=True)
            return c

        lax.fori_loop(0, _CHUNK // 2, body, 0)
        return carry

    lax.fori_loop(0, nblocks // _CHUNK, chunk_body, 0)
    plsc.subcore_barrier()
    pltpu.sync_copy(acc.at[pl.ds(r0, _RPT)], out_hbm.at[cid, pl.ds(r0, _RPT)])


def _make_spmm(split_edges):
    nblocks = _BLOCKS_HALF if split_edges else _BLOCKS_ALL
    return pl.kernel(
        functools.partial(_spmm_body, split_edges),
        out_type=jax.ShapeDtypeStruct((_NC, _ND, 128), jnp.float32),
        mesh=_sc_mesh,
        scratch_types=[
            pltpu.VMEM_SHARED((_ND, 128), jnp.float32),
            pltpu.VMEM((_CHUNK, _BLK), jnp.int32),
            pltpu.VMEM((_CHUNK, _BLK), jnp.int32),
            pltpu.VMEM((_BLK, 128), jnp.float32),
            pltpu.VMEM((_BLK, 128), jnp.float32),
            pltpu.SemaphoreType.DMA,
            pltpu.SemaphoreType.DMA,
        ],
    )


_spmm0_call = _make_spmm(True)
_spmm1_call = _make_spmm(False)


def _scale_body(h0_ref, h1_ref, x_ref, out_ref):
    dinv = lax.rsqrt(h0_ref[:, :1] + h1_ref[:, :1] + 1.0)
    out_ref[...] = x_ref[...] * dinv


_scale_call = pl.pallas_call(
    _scale_body,
    grid=(_ND // _TC_ROWS,),
    in_specs=[
        pl.BlockSpec((_TC_ROWS, 128), lambda i: (i, 0)),
        pl.BlockSpec((_TC_ROWS, 128), lambda i: (i, 0)),
        pl.BlockSpec((_TC_ROWS, 128), lambda i: (i, 0)),
    ],
    out_specs=pl.BlockSpec((_TC_ROWS, 128), lambda i: (i, 0)),
    out_shape=jax.ShapeDtypeStruct((_ND, 128), jnp.float32),
)


def _mid_body(h0_ref, h1_ref, a_ref, xt_ref, w1_ref, b1_ref, out_ref):
    dinv = lax.rsqrt(h0_ref[:, :1] + h1_ref[:, :1] + 1.0)
    agg = (a_ref[0] + a_ref[1] - xt_ref[...]) * dinv
    h = jnp.dot(agg, w1_ref[...], preferred_element_type=jnp.float32)
    h = jnp.maximum(h + b1_ref[...], 0.0) * dinv
    out_ref[0] = h[:, :128]
    out_ref[1] = h[:, 128:]


_mid_call = pl.pallas_call(
    _mid_body,
    grid=(_ND // _TC_ROWS,),
    in_specs=[
        pl.BlockSpec((_TC_ROWS, 128), lambda i: (i, 0)),
        pl.BlockSpec((_TC_ROWS, 128), lambda i: (i, 0)),
        pl.BlockSpec((_NC, _TC_ROWS, 128), lambda i: (0, i, 0)),
        pl.BlockSpec((_TC_ROWS, 128), lambda i: (i, 0)),
        pl.BlockSpec((128, 256), lambda i: (0, 0)),
        pl.BlockSpec((1, 256), lambda i: (0, 0)),
    ],
    out_specs=pl.BlockSpec((_NC, _TC_ROWS, 128), lambda i: (0, i, 0)),
    out_shape=jax.ShapeDtypeStruct((_NC, _ND, 128), jnp.float32),
)


def _out_body(h0_ref, h1_ref, a_ref, w2_ref, b2_ref, wl_ref, bl_ref, out_ref):
    dinv = lax.rsqrt(h0_ref[:, :1] + h1_ref[:, :1] + 1.0)
    agg = jnp.concatenate([a_ref[0], a_ref[1]], axis=1) * dinv
    h = jnp.dot(agg, w2_ref[...], preferred_element_type=jnp.float32)
    h = jnp.maximum(h + b2_ref[...], 0.0)
    out_ref[...] = (
        jnp.dot(h, wl_ref[...], preferred_element_type=jnp.float32)
        + bl_ref[...]
    )


_out_call = pl.pallas_call(
    _out_body,
    grid=(_ND // _TC_ROWS,),
    in_specs=[
        pl.BlockSpec((_TC_ROWS, 128), lambda i: (i, 0)),
        pl.BlockSpec((_TC_ROWS, 128), lambda i: (i, 0)),
        pl.BlockSpec((_NC, _TC_ROWS, 128), lambda i: (0, i, 0)),
        pl.BlockSpec((256, 256), lambda i: (0, 0)),
        pl.BlockSpec((1, 256), lambda i: (0, 0)),
        pl.BlockSpec((256, 128), lambda i: (0, 0)),
        pl.BlockSpec((1, 128), lambda i: (0, 0)),
    ],
    out_specs=pl.BlockSpec((_TC_ROWS, 128), lambda i: (i, 0)),
    out_shape=jax.ShapeDtypeStruct((_ND, 128), jnp.float32),
)


def kernel(x, edge_index, W1, b1, W2, b2, Wl, bl):
    n_out = Wl.shape[1]
    src = edge_index[0].astype(jnp.int32)
    dst = edge_index[1].astype(jnp.int32)
    pad_e = _EP - _E
    src_p = jnp.concatenate([src, jnp.zeros((pad_e,), jnp.int32)])
    dst_p = jnp.concatenate([dst, jnp.full((pad_e,), _N, jnp.int32)])
    src2d = src_p.reshape(_EP // _BLK, _BLK)
    dst2d = dst_p.reshape(_EP // _BLK, _BLK)

    zeros_hist = jnp.zeros((_ND, 128), jnp.float32)
    ones_blk = jnp.ones((_BLK, 128), jnp.float32)
    hist = _hist_call(dst2d, zeros_hist, ones_blk)
    h0, h1 = hist[0], hist[1]

    x_pad = jnp.pad(x, ((0, _ND - _N), (0, 0)))
    xt = _scale_call(h0, h1, x_pad)
    a0 = _spmm0_call(xt, src2d, dst2d)
    ht = _mid_call(h0, h1, a0, xt, W1, b1.reshape(1, -1))
    a1 = _spmm1_call(ht, src2d, dst2d)

    wl_pad = jnp.pad(Wl, ((0, 0), (0, 128 - n_out)))
    bl_pad = jnp.pad(bl, (0, 128 - n_out)).reshape(1, -1)
    out = _out_call(h0, h1, a1, W2, b2.reshape(1, -1), wl_pad, bl_pad)
    return out[:_N, :n_out]


# speedup vs baseline: 2.5206x; 2.5206x over previous
import functools

import jax
import jax.numpy as jnp
from jax import lax
from jax.experimental import pallas as pl
from jax.experimental.pallas import tpu as pltpu
from jax.experimental.pallas import tpu_sc as plsc

_N = 10000
_E = 320000
_ND = 10240
_EP = 327680
_NC = 2
_NT = 16
_BLK = 128
_RPT = _ND // _NT
_BLOCKS_ALL = _EP // (_NT * _BLK)
_BLOCKS_HALF = _EP // (_NC * _NT * _BLK)

_CHUNK = 16
_TC_ROWS = 640

_sc_mesh = plsc.VectorSubcoreMesh(core_axis_name="c", subcore_axis_name="s")


def _hist_body(dst_hbm, zeros_hbm, ones_hbm, out_hbm, acc, didx, ones_v,
               sem0, sem1):
    cid = lax.axis_index("c")
    sid = lax.axis_index("s")
    r0 = sid * _RPT
    pltpu.sync_copy(zeros_hbm.at[pl.ds(r0, _RPT)], acc.at[pl.ds(r0, _RPT)])
    pltpu.sync_copy(ones_hbm, ones_v)
    tid = cid * _NT + sid
    blk0 = tid * _BLOCKS_HALF
    pltpu.sync_copy(dst_hbm.at[pl.ds(blk0, _BLOCKS_HALF)], didx)
    plsc.subcore_barrier()

    def body(i, carry):
        @pl.when(i > 0)
        def _():
            pltpu.make_async_copy(ones_v, acc.at[didx.at[0]], sem0).wait()
            pltpu.make_async_copy(ones_v, acc.at[didx.at[0]], sem1).wait()

        pltpu.async_copy(ones_v, acc.at[didx.at[2 * i]], sem0, add=True)
        pltpu.async_copy(ones_v, acc.at[didx.at[2 * i + 1]], sem1, add=True)
        return carry

    lax.fori_loop(0, _BLOCKS_HALF // 2, body, 0)
    pltpu.make_async_copy(ones_v, acc.at[didx.at[0]], sem0).wait()
    pltpu.make_async_copy(ones_v, acc.at[didx.at[0]], sem1).wait()
    plsc.subcore_barrier()
    pltpu.sync_copy(acc.at[pl.ds(r0, _RPT)], out_hbm.at[cid, pl.ds(r0, _RPT)])


_hist_call = pl.kernel(
    _hist_body,
    out_type=jax.ShapeDtypeStruct((_NC, _ND, 128), jnp.float32),
    mesh=_sc_mesh,
    scratch_types=[
        pltpu.VMEM_SHARED((_ND, 128), jnp.float32),
        pltpu.VMEM((_BLOCKS_HALF, _BLK), jnp.int32),
        pltpu.VMEM((_BLK, 128), jnp.float32),
        pltpu.SemaphoreType.DMA,
        pltpu.SemaphoreType.DMA,
    ],
)


def _spmm_body(split_edges, tables_hbm, src_hbm, dst_hbm, out_hbm, acc,
               sidx, didx, rows0, rows1, gsem0, gsem1):
    cid = lax.axis_index("c")
    sid = lax.axis_index("s")
    r0 = sid * _RPT
    if split_edges:
        table = tables_hbm
        nblocks = _BLOCKS_HALF
        blk0 = (cid * _NT + sid) * _BLOCKS_HALF
    else:
        table = tables_hbm.at[cid]
        nblocks = _BLOCKS_ALL
        blk0 = sid * _BLOCKS_ALL
    pltpu.sync_copy(table.at[pl.ds(r0, _RPT)], acc.at[pl.ds(r0, _RPT)])
    plsc.subcore_barrier()

    def chunk_body(ci, carry):
        base = blk0 + ci * _CHUNK
        pltpu.sync_copy(src_hbm.at[pl.ds(base, _CHUNK)], sidx)
        pltpu.sync_copy(dst_hbm.at[pl.ds(base, _CHUNK)], didx)
        pltpu.async_copy(table.at[sidx.at[0]], rows0, gsem0)

        def body(i, c):
            b = 2 * i
            pltpu.async_copy(table.at[sidx.at[b + 1]], rows1, gsem1)
            pltpu.make_async_copy(table.at[sidx.at[b]], rows0, gsem0).wait()
            pltpu.sync_copy(rows0, acc.at[didx.at[b]], add=True)

            @pl.when(b + 2 < _CHUNK)
            def _():
                pltpu.async_copy(table.at[sidx.at[b + 2]], rows0, gsem0)

            pltpu.make_async_copy(table.at[sidx.at[b + 1]], rows1,
                                  gsem1).wait()
            pltpu.sync_copy(rows1, acc.at[didx.at[b + 1]], add=True)
            return c

        lax.fori_loop(0, _CHUNK // 2, body, 0)
        return carry

    lax.fori_loop(0, nblocks // _CHUNK, chunk_body, 0)
    plsc.subcore_barrier()
    pltpu.sync_copy(acc.at[pl.ds(r0, _RPT)], out_hbm.at[cid, pl.ds(r0, _RPT)])


def _make_spmm(split_edges):
    nblocks = _BLOCKS_HALF if split_edges else _BLOCKS_ALL
    return pl.kernel(
        functools.partial(_spmm_body, split_edges),
        out_type=jax.ShapeDtypeStruct((_NC, _ND, 128), jnp.float32),
        mesh=_sc_mesh,
        scratch_types=[
            pltpu.VMEM_SHARED((_ND, 128), jnp.float32),
            pltpu.VMEM((_CHUNK, _BLK), jnp.int32),
            pltpu.VMEM((_CHUNK, _BLK), jnp.int32),
            pltpu.VMEM((_BLK, 128), jnp.float32),
            pltpu.VMEM((_BLK, 128), jnp.float32),
            pltpu.SemaphoreType.DMA,
            pltpu.SemaphoreType.DMA,
        ],
    )


_spmm0_call = _make_spmm(True)
_spmm1_call = _make_spmm(False)


def _scale_body(h0_ref, h1_ref, x_ref, out_ref):
    dinv = lax.rsqrt(h0_ref[:, :1] + h1_ref[:, :1] + 1.0)
    out_ref[...] = x_ref[...] * dinv


_scale_call = pl.pallas_call(
    _scale_body,
    grid=(_ND // _TC_ROWS,),
    in_specs=[
        pl.BlockSpec((_TC_ROWS, 128), lambda i: (i, 0)),
        pl.BlockSpec((_TC_ROWS, 128), lambda i: (i, 0)),
        pl.BlockSpec((_TC_ROWS, 128), lambda i: (i, 0)),
    ],
    out_specs=pl.BlockSpec((_TC_ROWS, 128), lambda i: (i, 0)),
    out_shape=jax.ShapeDtypeStruct((_ND, 128), jnp.float32),
)


def _mid_body(h0_ref, h1_ref, a_ref, xt_ref, w1_ref, b1_ref, out_ref):
    dinv = lax.rsqrt(h0_ref[:, :1] + h1_ref[:, :1] + 1.0)
    agg = (a_ref[0] + a_ref[1] - xt_ref[...]) * dinv
    h = jnp.dot(agg, w1_ref[...], preferred_element_type=jnp.float32)
    h = jnp.maximum(h + b1_ref[...], 0.0) * dinv
    out_ref[0] = h[:, :128]
    out_ref[1] = h[:, 128:]


_mid_call = pl.pallas_call(
    _mid_body,
    grid=(_ND // _TC_ROWS,),
    in_specs=[
        pl.BlockSpec((_TC_ROWS, 128), lambda i: (i, 0)),
        pl.BlockSpec((_TC_ROWS, 128), lambda i: (i, 0)),
        pl.BlockSpec((_NC, _TC_ROWS, 128), lambda i: (0, i, 0)),
        pl.BlockSpec((_TC_ROWS, 128), lambda i: (i, 0)),
        pl.BlockSpec((128, 256), lambda i: (0, 0)),
        pl.BlockSpec((1, 256), lambda i: (0, 0)),
    ],
    out_specs=pl.BlockSpec((_NC, _TC_ROWS, 128), lambda i: (0, i, 0)),
    out_shape=jax.ShapeDtypeStruct((_NC, _ND, 128), jnp.float32),
)


def _out_body(h0_ref, h1_ref, a_ref, w2_ref, b2_ref, wl_ref, bl_ref, out_ref):
    dinv = lax.rsqrt(h0_ref[:, :1] + h1_ref[:, :1] + 1.0)
    agg = jnp.concatenate([a_ref[0], a_ref[1]], axis=1) * dinv
    h = jnp.dot(agg, w2_ref[...], preferred_element_type=jnp.float32)
    h = jnp.maximum(h + b2_ref[...], 0.0)
    out_ref[...] = (
        jnp.dot(h, wl_ref[...], preferred_element_type=jnp.float32)
        + bl_ref[...]
    )


_out_call = pl.pallas_call(
    _out_body,
    grid=(_ND // _TC_ROWS,),
    in_specs=[
        pl.BlockSpec((_TC_ROWS, 128), lambda i: (i, 0)),
        pl.BlockSpec((_TC_ROWS, 128), lambda i: (i, 0)),
        pl.BlockSpec((_NC, _TC_ROWS, 128), lambda i: (0, i, 0)),
        pl.BlockSpec((256, 256), lambda i: (0, 0)),
        pl.BlockSpec((1, 256), lambda i: (0, 0)),
        pl.BlockSpec((256, 128), lambda i: (0, 0)),
        pl.BlockSpec((1, 128), lambda i: (0, 0)),
    ],
    out_specs=pl.BlockSpec((_TC_ROWS, 128), lambda i: (i, 0)),
    out_shape=jax.ShapeDtypeStruct((_ND, 128), jnp.float32),
)


def kernel(x, edge_index, W1, b1, W2, b2, Wl, bl):
    n_out = Wl.shape[1]
    src = edge_index[0].astype(jnp.int32)
    dst = edge_index[1].astype(jnp.int32)
    pad_e = _EP - _E
    spread = jnp.arange(pad_e, dtype=jnp.int32) % 128
    src_p = jnp.concatenate([src, spread])
    dst_p = jnp.concatenate([dst, _N + spread])
    src2d = src_p.reshape(_EP // _BLK, _BLK)
    dst2d = dst_p.reshape(_EP // _BLK, _BLK)

    zeros_hist = jnp.zeros((_ND, 128), jnp.float32)
    ones_blk = jnp.ones((_BLK, 128), jnp.float32)
    hist = _hist_call(dst2d, zeros_hist, ones_blk)
    h0, h1 = hist[0], hist[1]

    x_pad = jnp.pad(x, ((0, _ND - _N), (0, 0)))
    xt = _scale_call(h0, h1, x_pad)
    a0 = _spmm0_call(xt, src2d, dst2d)
    ht = _mid_call(h0, h1, a0, xt, W1, b1.reshape(1, -1))
    a1 = _spmm1_call(ht, src2d, dst2d)

    wl_pad = jnp.pad(Wl, ((0, 0), (0, 128 - n_out)))
    bl_pad = jnp.pad(bl, (0, 128 - n_out)).reshape(1, -1)
    out = _out_call(h0, h1, a1, W2, b2.reshape(1, -1), wl_pad, bl_pad)
    return out[:_N, :n_out]


# speedup vs baseline: 2.9071x; 1.1533x over previous
import functools

import jax
import jax.numpy as jnp
from jax import lax
from jax.experimental import pallas as pl
from jax.experimental.pallas import tpu as pltpu
from jax.experimental.pallas import tpu_sc as plsc

_N = 10000
_E = 320000
_ND = 10240
_EP = 327680
_NC = 2
_NT = 16
_BLK = 128
_RPT = _ND // _NT
_BLOCKS_ALL = _EP // (_NT * _BLK)
_BLOCKS_HALF = _EP // (_NC * _NT * _BLK)

_CHUNK = 32
_TC_ROWS = 2000

_sc_mesh = plsc.VectorSubcoreMesh(core_axis_name="c", subcore_axis_name="s")


def _hist_body(dst_hbm, zeros_hbm, ones_hbm, out_hbm, acc, didx, ones_v,
               sem0, sem1):
    cid = lax.axis_index("c")
    sid = lax.axis_index("s")
    r0 = sid * _RPT
    pltpu.sync_copy(zeros_hbm.at[pl.ds(r0, _RPT)], acc.at[pl.ds(r0, _RPT)])
    pltpu.sync_copy(ones_hbm, ones_v)
    tid = cid * _NT + sid
    blk0 = tid * _BLOCKS_HALF
    pltpu.sync_copy(dst_hbm.at[pl.ds(blk0, _BLOCKS_HALF)], didx)
    plsc.subcore_barrier()

    def body(i, carry):
        @pl.when(i > 0)
        def _():
            pltpu.make_async_copy(ones_v, acc.at[didx.at[0]], sem0).wait()
            pltpu.make_async_copy(ones_v, acc.at[didx.at[0]], sem1).wait()

        pltpu.async_copy(ones_v, acc.at[didx.at[2 * i]], sem0, add=True)
        pltpu.async_copy(ones_v, acc.at[didx.at[2 * i + 1]], sem1, add=True)
        return carry

    lax.fori_loop(0, _BLOCKS_HALF // 2, body, 0)
    pltpu.make_async_copy(ones_v, acc.at[didx.at[0]], sem0).wait()
    pltpu.make_async_copy(ones_v, acc.at[didx.at[0]], sem1).wait()
    plsc.subcore_barrier()
    pltpu.sync_copy(acc.at[pl.ds(r0, _RPT)], out_hbm.at[cid, pl.ds(r0, _RPT)])


_hist_call = pl.kernel(
    _hist_body,
    out_type=jax.ShapeDtypeStruct((_NC, _ND, 128), jnp.float32),
    mesh=_sc_mesh,
    scratch_types=[
        pltpu.VMEM_SHARED((_ND, 128), jnp.float32),
        pltpu.VMEM((_BLOCKS_HALF, _BLK), jnp.int32),
        pltpu.VMEM((_BLK, 128), jnp.float32),
        pltpu.SemaphoreType.DMA,
        pltpu.SemaphoreType.DMA,
    ],
)


def _spmm_body(split_edges, tables_hbm, src_hbm, dst_hbm, out_hbm, acc,
               sidx, didx, rows0, rows1, gsem0, gsem1):
    cid = lax.axis_index("c")
    sid = lax.axis_index("s")
    r0 = sid * _RPT
    if split_edges:
        table = tables_hbm
        nblocks = _BLOCKS_HALF
        blk0 = (cid * _NT + sid) * _BLOCKS_HALF
    else:
        table = tables_hbm.at[cid]
        nblocks = _BLOCKS_ALL
        blk0 = sid * _BLOCKS_ALL
    pltpu.sync_copy(table.at[pl.ds(r0, _RPT)], acc.at[pl.ds(r0, _RPT)])
    plsc.subcore_barrier()

    def chunk_body(ci, carry):
        base = blk0 + ci * _CHUNK
        pltpu.sync_copy(src_hbm.at[pl.ds(base, _CHUNK)], sidx)
        pltpu.sync_copy(dst_hbm.at[pl.ds(base, _CHUNK)], didx)
        pltpu.async_copy(table.at[sidx.at[0]], rows0, gsem0)

        def body(i, c):
            b = 2 * i
            pltpu.async_copy(table.at[sidx.at[b + 1]], rows1, gsem1)
            pltpu.make_async_copy(table.at[sidx.at[b]], rows0, gsem0).wait()
            pltpu.sync_copy(rows0, acc.at[didx.at[b]], add=True)

            @pl.when(b + 2 < _CHUNK)
            def _():
                pltpu.async_copy(table.at[sidx.at[b + 2]], rows0, gsem0)

            pltpu.make_async_copy(table.at[sidx.at[b + 1]], rows1,
                                  gsem1).wait()
            pltpu.sync_copy(rows1, acc.at[didx.at[b + 1]], add=True)
            return c

        lax.fori_loop(0, _CHUNK // 2, body, 0)
        return carry

    lax.fori_loop(0, nblocks // _CHUNK, chunk_body, 0)
    plsc.subcore_barrier()
    pltpu.sync_copy(acc.at[pl.ds(r0, _RPT)], out_hbm.at[cid, pl.ds(r0, _RPT)])


def _make_spmm(split_edges):
    nblocks = _BLOCKS_HALF if split_edges else _BLOCKS_ALL
    return pl.kernel(
        functools.partial(_spmm_body, split_edges),
        out_type=jax.ShapeDtypeStruct((_NC, _ND, 128), jnp.float32),
        mesh=_sc_mesh,
        scratch_types=[
            pltpu.VMEM_SHARED((_ND, 128), jnp.float32),
            pltpu.VMEM((_CHUNK, _BLK), jnp.int32),
            pltpu.VMEM((_CHUNK, _BLK), jnp.int32),
            pltpu.VMEM((_BLK, 128), jnp.float32),
            pltpu.VMEM((_BLK, 128), jnp.float32),
            pltpu.SemaphoreType.DMA,
            pltpu.SemaphoreType.DMA,
        ],
    )


_spmm0_call = _make_spmm(True)
_spmm1_call = _make_spmm(False)


def _scale_body(h0_ref, h1_ref, x_ref, out_ref):
    dinv = lax.rsqrt(h0_ref[0, :, :1] + h1_ref[0, :, :1] + 1.0)
    out_ref[...] = x_ref[...] * dinv


_scale_call = pl.pallas_call(
    _scale_body,
    grid=(_N // _TC_ROWS,),
    in_specs=[
        pl.BlockSpec((1, _TC_ROWS, 128), lambda i: (0, i, 0)),
        pl.BlockSpec((1, _TC_ROWS, 128), lambda i: (1, i, 0)),
        pl.BlockSpec((_TC_ROWS, 128), lambda i: (i, 0)),
    ],
    out_specs=pl.BlockSpec((_TC_ROWS, 128), lambda i: (i, 0)),
    out_shape=jax.ShapeDtypeStruct((_ND, 128), jnp.float32),
)


def _mid_body(h0_ref, h1_ref, a_ref, xt_ref, w1_ref, b1_ref, out_ref):
    dinv = lax.rsqrt(h0_ref[0, :, :1] + h1_ref[0, :, :1] + 1.0)
    agg = (a_ref[0] + a_ref[1] - xt_ref[...]) * dinv
    h = jnp.dot(agg, w1_ref[...], preferred_element_type=jnp.float32)
    h = jnp.maximum(h + b1_ref[...], 0.0) * dinv
    out_ref[0] = h[:, :128]
    out_ref[1] = h[:, 128:]


_mid_call = pl.pallas_call(
    _mid_body,
    grid=(_N // _TC_ROWS,),
    in_specs=[
        pl.BlockSpec((1, _TC_ROWS, 128), lambda i: (0, i, 0)),
        pl.BlockSpec((1, _TC_ROWS, 128), lambda i: (1, i, 0)),
        pl.BlockSpec((_NC, _TC_ROWS, 128), lambda i: (0, i, 0)),
        pl.BlockSpec((_TC_ROWS, 128), lambda i: (i, 0)),
        pl.BlockSpec((128, 256), lambda i: (0, 0)),
        pl.BlockSpec((1, 256), lambda i: (0, 0)),
    ],
    out_specs=pl.BlockSpec((_NC, _TC_ROWS, 128), lambda i: (0, i, 0)),
    out_shape=jax.ShapeDtypeStruct((_NC, _ND, 128), jnp.float32),
)


def _out_body(h0_ref, h1_ref, a_ref, w2_ref, b2_ref, wl_ref, bl_ref, out_ref):
    dinv = lax.rsqrt(h0_ref[0, :, :1] + h1_ref[0, :, :1] + 1.0)
    agg = jnp.concatenate([a_ref[0], a_ref[1]], axis=1) * dinv
    h = jnp.dot(agg, w2_ref[...], preferred_element_type=jnp.float32)
    h = jnp.maximum(h + b2_ref[...], 0.0)
    out_ref[...] = (
        jnp.dot(h, wl_ref[...], preferred_element_type=jnp.float32)[:, :18]
        + bl_ref[...]
    )


_out_call = pl.pallas_call(
    _out_body,
    grid=(_N // _TC_ROWS,),
    in_specs=[
        pl.BlockSpec((1, _TC_ROWS, 128), lambda i: (0, i, 0)),
        pl.BlockSpec((1, _TC_ROWS, 128), lambda i: (1, i, 0)),
        pl.BlockSpec((_NC, _TC_ROWS, 128), lambda i: (0, i, 0)),
        pl.BlockSpec((256, 256), lambda i: (0, 0)),
        pl.BlockSpec((1, 256), lambda i: (0, 0)),
        pl.BlockSpec((256, 128), lambda i: (0, 0)),
        pl.BlockSpec((1, 18), lambda i: (0, 0)),
    ],
    out_specs=pl.BlockSpec((_TC_ROWS, 18), lambda i: (i, 0)),
    out_shape=jax.ShapeDtypeStruct((_N, 18), jnp.float32),
)


def kernel(x, edge_index, W1, b1, W2, b2, Wl, bl):
    n_out = Wl.shape[1]
    src = edge_index[0].astype(jnp.int32)
    dst = edge_index[1].astype(jnp.int32)
    pad_e = _EP - _E
    spread = jnp.arange(pad_e, dtype=jnp.int32) % 128
    src_p = jnp.concatenate([src, spread])
    dst_p = jnp.concatenate([dst, _N + spread])
    src2d = src_p.reshape(_EP // _BLK, _BLK)
    dst2d = dst_p.reshape(_EP // _BLK, _BLK)

    zeros_hist = jnp.zeros((_ND, 128), jnp.float32)
    ones_blk = jnp.ones((_BLK, 128), jnp.float32)
    hist = _hist_call(dst2d, zeros_hist, ones_blk)

    xt = _scale_call(hist, hist, x)
    a0 = _spmm0_call(xt, src2d, dst2d)
    ht = _mid_call(hist, hist, a0, xt, W1, b1.reshape(1, -1))
    a1 = _spmm1_call(ht, src2d, dst2d)

    wl_pad = jnp.pad(Wl, ((0, 0), (0, 128 - n_out)))
    out = _out_call(hist, hist, a1, W2, b2.reshape(1, -1), wl_pad,
                    bl.reshape(1, -1))
    return out
